# baseline (speedup 1.0000x reference)
import functools

import jax
import jax.numpy as jnp
from jax import lax
from jax.experimental import pallas as pl
from jax.experimental.pallas import tpu as pltpu
from jax.experimental.pallas import tpu_sc as plsc

D = 512
TOTAL = 16384
SEG = 32
NSEG = TOTAL // SEG
LANES = 16

_mesh = plsc.VectorSubcoreMesh(core_axis_name="c", subcore_axis_name="s")
_NW = _mesh.num_cores * _mesh.num_subcores
_ROWS_PER_W = D // _NW
_WORK = _ROWS_PER_W * TOTAL
_OUT_W = _ROWS_PER_W * NSEG
_NCHUNK = 8
_CHUNK = _WORK // _NCHUNK
_GROUPS = _CHUNK // (LANES * SEG)


@functools.partial(
    pl.kernel,
    out_type=jax.ShapeDtypeStruct((_NW, _OUT_W), jnp.float32),
    mesh=_mesh,
    scratch_types=[
        pltpu.VMEM((_CHUNK,), jnp.float32),
        pltpu.VMEM((_CHUNK,), jnp.float32),
        pltpu.VMEM((_OUT_W,), jnp.float32),
        pltpu.SemaphoreType.DMA,
        pltpu.SemaphoreType.DMA,
    ],
    compiler_params=pltpu.CompilerParams(needs_layout_passes=False),
)
def _prod_at(x_hbm, out_hbm, buf0, buf1, out_buf, sem0, sem1):
    wid = lax.axis_index("s") * _mesh.num_cores + lax.axis_index("c")
    lane = lax.iota(jnp.int32, LANES)
    idx = [lane * SEG + k for k in range(SEG)]
    bufs = (buf0, buf1)
    sems = (sem0, sem1)

    copies = [pltpu.async_copy(
        x_hbm.at[wid, pl.ds(0, _CHUNK)], bufs[0], sems[0])]
    for c in range(_NCHUNK):
        p = c % 2
        copies[c].wait()
        if c + 1 < _NCHUNK:
            copies.append(pltpu.async_copy(
                x_hbm.at[wid, pl.ds((c + 1) * _CHUNK, _CHUNK)],
                bufs[1 - p], sems[1 - p]))

        def group_body(g, carry, p=p, c=c):
            off = g * (LANES * SEG)
            vals = [plsc.load_gather(bufs[p], [idx[k] + off])
                    for k in range(SEG)]
            while len(vals) > 1:
                vals = [vals[i] * vals[i + 1] for i in range(0, len(vals), 2)]
            plsc.store_scatter(
                out_buf, [lane + (c * _GROUPS + g) * LANES], vals[0])
            return carry

        lax.fori_loop(0, _GROUPS, group_body, 0)

    pltpu.sync_copy(out_buf, out_hbm.at[wid])


def kernel(x):
    out = _prod_at(x.reshape(_NW, _WORK))
    return out.reshape(D, NSEG)


# speedup vs baseline: 2.0982x; 2.0982x over previous
import functools

import jax
import jax.numpy as jnp
from jax import lax
from jax.experimental import pallas as pl
from jax.experimental.pallas import tpu as pltpu
from jax.experimental.pallas import tpu_sc as plsc

D = 512
TOTAL = 16384
SEG = 32
NSEG = TOTAL // SEG
LANES = 16

_mesh = plsc.VectorSubcoreMesh(core_axis_name="c", subcore_axis_name="s")
_NW = _mesh.num_cores * _mesh.num_subcores
_ROWS_PER_W = D // _NW
_WORK = _ROWS_PER_W * TOTAL
_OUT_W = _ROWS_PER_W * NSEG
_NCHUNK = 8
_CHUNK = _WORK // _NCHUNK
_GROUPS = _CHUNK // (LANES * SEG)


@functools.partial(
    pl.kernel,
    out_type=jax.ShapeDtypeStruct((_NW, _OUT_W), jnp.float32),
    mesh=_mesh,
    scratch_types=[
        pltpu.VMEM((_CHUNK,), jnp.float32),
        pltpu.VMEM((_CHUNK,), jnp.float32),
        pltpu.VMEM((_OUT_W,), jnp.float32),
        pltpu.SemaphoreType.DMA,
        pltpu.SemaphoreType.DMA,
    ],
    compiler_params=pltpu.CompilerParams(needs_layout_passes=False),
)
def _prod_at(x_hbm, out_hbm, buf0, buf1, out_buf, sem0, sem1):
    wid = lax.axis_index("s") * _mesh.num_cores + lax.axis_index("c")
    lane = lax.iota(jnp.int32, LANES)
    idx = [lane * SEG + ((lane + k) & (SEG - 1)) for k in range(SEG)]
    bufs = (buf0, buf1)
    sems = (sem0, sem1)

    copies = [pltpu.async_copy(
        x_hbm.at[wid, pl.ds(0, _CHUNK)], bufs[0], sems[0])]
    for c in range(_NCHUNK):
        p = c % 2
        copies[c].wait()
        if c + 1 < _NCHUNK:
            copies.append(pltpu.async_copy(
                x_hbm.at[wid, pl.ds((c + 1) * _CHUNK, _CHUNK)],
                bufs[1 - p], sems[1 - p]))

        def group_body(g, carry, p=p, c=c):
            off = g * (LANES * SEG)
            vals = [plsc.load_gather(bufs[p], [idx[k] + off])
                    for k in range(SEG)]
            while len(vals) > 1:
                vals = [vals[i] * vals[i + 1] for i in range(0, len(vals), 2)]
            plsc.store_scatter(
                out_buf, [lane + (c * _GROUPS + g) * LANES], vals[0])
            return carry

        lax.fori_loop(0, _GROUPS, group_body, 0)

    pltpu.sync_copy(out_buf, out_hbm.at[wid])


def kernel(x):
    out = _prod_at(x.reshape(_NW, _WORK))
    return out.reshape(D, NSEG)


# speedup vs baseline: 4.0794x; 1.9442x over previous
import functools

import jax
import jax.numpy as jnp
from jax import lax
from jax.experimental import pallas as pl
from jax.experimental.pallas import tpu as pltpu
from jax.experimental.pallas import tpu_sc as plsc

D = 512
TOTAL = 16384
SEG = 32
NSEG = TOTAL // SEG
LANES = 16
GSIZE = LANES * SEG

_mesh = plsc.VectorSubcoreMesh(core_axis_name="c", subcore_axis_name="s")
_NW = _mesh.num_cores * _mesh.num_subcores
_ROWS_PER_W = D // _NW
_NBUF = 4


@functools.partial(
    pl.kernel,
    out_type=jax.ShapeDtypeStruct((D, NSEG), jnp.float32),
    mesh=_mesh,
    scratch_types=[
        pltpu.VMEM((TOTAL,), jnp.float32),
        pltpu.VMEM((TOTAL,), jnp.float32),
        pltpu.VMEM((TOTAL,), jnp.float32),
        pltpu.VMEM((TOTAL,), jnp.float32),
        pltpu.VMEM((_ROWS_PER_W, NSEG), jnp.float32),
        pltpu.SemaphoreType.DMA,
        pltpu.SemaphoreType.DMA,
        pltpu.SemaphoreType.DMA,
        pltpu.SemaphoreType.DMA,
    ],
    compiler_params=pltpu.CompilerParams(needs_layout_passes=False),
)
def _prod_at(x_hbm, out_hbm, b0, b1, b2, b3, out_buf, s0, s1, s2, s3):
    wid = lax.axis_index("s") * _mesh.num_cores + lax.axis_index("c")
    row0 = wid * _ROWS_PER_W
    lane = lax.iota(jnp.int32, LANES)
    idx = [lane * SEG + ((lane + k) & (SEG - 1)) for k in range(SEG)]
    bufs = (b0, b1, b2, b3)
    sems = (s0, s1, s2, s3)

    copies = [pltpu.async_copy(x_hbm.at[row0 + r], bufs[r], sems[r])
              for r in range(_NBUF)]
    for r in range(_ROWS_PER_W):
        p = r % _NBUF
        copies[r].wait()

        def group_body(g, carry, p=p, r=r):
            off = g * GSIZE
            vals = [plsc.load_gather(bufs[p], [idx[k] + off])
                    for k in range(SEG)]
            while len(vals) > 1:
                vals = [vals[i] * vals[i + 1] for i in range(0, len(vals), 2)]
            plsc.store_scatter(out_buf, [lane * 0 + r, lane + g * LANES],
                               vals[0])
            return carry

        lax.fori_loop(0, NSEG // LANES, group_body, 0)
        if r + _NBUF < _ROWS_PER_W:
            copies.append(pltpu.async_copy(
                x_hbm.at[row0 + r + _NBUF], bufs[p], sems[p]))

    pltpu.sync_copy(out_buf, out_hbm.at[pl.ds(row0, _ROWS_PER_W), :])


def kernel(x):
    return _prod_at(x)
